# baseline (speedup 1.0000x reference)
import functools

import jax
import jax.numpy as jnp
from jax import lax
from jax.experimental import pallas as pl
from jax.experimental.pallas import tpu as pltpu
from jax.experimental.pallas import tpu_sc as plsc

N = 1000
NBP = 1024
TBL = N * NBP
NUM_SAMPLES = 2_000_000
NC, NS, L = 2, 16, 16
NW = NC * NS
NB = NUM_SAMPLES // 128
BPW = NB // NW
REM = NB - BPW * NW
CHB = 32
N_FULL = BPW // CHB
TAILB = BPW - N_FULL * CHB + REM
CH = CHB * 128
CBL = TAILB * 128
ZB = 8000
Z_SPAN = TBL // NS
DUMMY = (7 << 10) | 127


def _hist_body(inp, out, table, inbuf0, inbuf1, cbuf0, cbuf1, ones1d,
               cbuf_l, ones_l, zbuf, sem_in0, sem_in1, sem_s0, sem_s1,
               sem_l, sem_z):
    cid = lax.axis_index("c")
    sid = lax.axis_index("s")
    wid = cid * NS + sid
    base = wid * BPW
    iota = lax.iota(jnp.int32, L)
    zeros16 = jnp.zeros((L,), jnp.float32)
    ones16 = jnp.ones((L,), jnp.float32)
    dummy16 = jnp.full((L,), DUMMY, jnp.int32)
    inbufs = [inbuf0, inbuf1]
    cbufs = [cbuf0, cbuf1]
    sem_in = [sem_in0, sem_in1]
    sem_s = [sem_s0, sem_s1]

    def fill_z(i, _):
        zbuf[pl.ds(i * L, L)] = zeros16
        return _

    lax.fori_loop(0, ZB // L, fill_z, None)

    def fill_ones(i, _):
        ones1d[pl.ds(i * L, L)] = ones16
        return _

    lax.fori_loop(0, CH // L, fill_ones, None)

    def fill_ones_l(i, _):
        ones_l[pl.ds(i * L, L)] = ones16
        return _

    lax.fori_loop(0, CBL // L, fill_ones_l, None)

    zcopies = [
        pltpu.async_copy(zbuf, table.at[pl.ds(sid * Z_SPAN + j * ZB, ZB)], sem_z)
        for j in range(Z_SPAN // ZB)
    ]
    pending_in = pltpu.async_copy(inp.at[pl.ds(base, CHB)], inbufs[0], sem_in[0])
    for c in zcopies:
        c.wait()
    plsc.subcore_barrier()

    def tiled_bin(va, vb):
        return (
            ((va >> 3) << 13)
            | ((vb >> 7) << 10)
            | ((va & 7) << 7)
            | (vb & 127)
        )

    def compute_blocks(src, cdst):
        def blk_body(blk, _):
            for gg in range(8):
                va = src[blk, 0, pl.ds(gg * L, L)]
                vb = src[blk, 1, pl.ds(gg * L, L)]
                cdst[pl.ds(blk * 128 + gg * L, L)] = tiled_bin(va, vb)
            return _

        lax.fori_loop(0, CHB, blk_body, None)

    nreal = jnp.where(wid == NW - 1, TAILB, TAILB - REM)

    def compute_tail(src):
        def blk_body_l(blk, _):
            for gg in range(8):
                va = src[blk, 0, pl.ds(gg * L, L)]
                vb = src[blk, 1, pl.ds(gg * L, L)]
                c = jnp.where(blk < nreal, tiled_bin(va, vb), dummy16)
                cbuf_l[pl.ds(blk * 128 + gg * L, L)] = c
            return _

        lax.fori_loop(0, TAILB, blk_body_l, None)

    prev_scat = None
    for ch in range(N_FULL + 1):
        cur = ch & 1
        nxt = 1 - cur
        next_in = None
        if ch + 1 < N_FULL:
            next_in = pltpu.async_copy(
                inp.at[pl.ds(base + (ch + 1) * CHB, CHB)], inbufs[nxt], sem_in[nxt]
            )
        elif ch + 1 == N_FULL:
            next_in = pltpu.async_copy(
                inp.at[pl.ds(base + N_FULL * CHB, TAILB)],
                inbufs[nxt].at[pl.ds(0, TAILB)],
                sem_in[nxt],
            )
        pending_in.wait()
        if ch < N_FULL:
            compute_blocks(inbufs[cur], cbufs[cur])
            if prev_scat is not None:
                prev_scat.wait()
            prev_scat = pltpu.async_copy(
                ones1d, table.at[cbufs[cur]], sem_s[cur], add=True
            )
        else:
            compute_tail(inbufs[cur])
            if prev_scat is not None:
                prev_scat.wait()
            prev_scat = pltpu.async_copy(ones_l, table.at[cbuf_l], sem_l, add=True)
        pending_in = next_in

    prev_scat.wait()

    plsc.subcore_barrier()
    pltpu.sync_copy(
        table.at[pl.ds(sid * Z_SPAN, Z_SPAN)],
        out.at[pl.ds(cid * TBL + sid * Z_SPAN, Z_SPAN)],
    )


_sc_hist = functools.partial(
    pl.kernel,
    out_type=jax.ShapeDtypeStruct((NC * TBL,), jnp.float32),
    mesh=plsc.VectorSubcoreMesh(core_axis_name="c", subcore_axis_name="s"),
    compiler_params=pltpu.CompilerParams(needs_layout_passes=False),
    scratch_types=[
        pltpu.VMEM_SHARED((TBL,), jnp.float32),
        pltpu.VMEM((CHB, 2, 128), jnp.int32),
        pltpu.VMEM((CHB, 2, 128), jnp.int32),
        pltpu.VMEM((CH,), jnp.int32),
        pltpu.VMEM((CH,), jnp.int32),
        pltpu.VMEM((CH,), jnp.float32),
        pltpu.VMEM((CBL,), jnp.int32),
        pltpu.VMEM((CBL,), jnp.float32),
        pltpu.VMEM((ZB,), jnp.float32),
        pltpu.SemaphoreType.DMA,
        pltpu.SemaphoreType.DMA,
        pltpu.SemaphoreType.DMA,
        pltpu.SemaphoreType.DMA,
        pltpu.SemaphoreType.DMA,
        pltpu.SemaphoreType.DMA,
    ],
)(_hist_body)


def _finalize_body(p_ref, o_ref, a_ref):
    s = p_ref[0] + p_ref[1]
    v = s[:, :N]
    rs = jnp.sum(v, axis=1, keepdims=True)
    a_ref[...] = rs * (1.0 / NUM_SAMPLES)
    o_ref[...] = v / jnp.maximum(rs, 1.0)


_FROWS = 200


def _finalize(p3):
    return pl.pallas_call(
        _finalize_body,
        grid=(N // _FROWS,),
        in_specs=[pl.BlockSpec((NC, _FROWS, NBP), lambda i: (0, i, 0))],
        out_specs=[
            pl.BlockSpec((_FROWS, N), lambda i: (i, 0)),
            pl.BlockSpec((_FROWS, 1), lambda i: (i, 0)),
        ],
        out_shape=[
            jax.ShapeDtypeStruct((N, N), jnp.float32),
            jax.ShapeDtypeStruct((N, 1), jnp.float32),
        ],
    )(p3)


def kernel(inputs, vals):
    del vals
    view = inputs.reshape(NB, 128, 2).transpose(0, 2, 1)
    part = _sc_hist(view)
    p3 = (
        part.reshape(NC, N // 8, 8, NBP // 128, 128)
        .transpose(0, 1, 3, 2, 4)
        .reshape(NC, N, NBP)
    )
    pi_b_a, pi_a = _finalize(p3)
    return pi_a.reshape(N), pi_b_a


# speedup vs baseline: 1.0053x; 1.0053x over previous
import functools

import jax
import jax.numpy as jnp
from jax import lax
from jax.experimental import pallas as pl
from jax.experimental.pallas import tpu as pltpu
from jax.experimental.pallas import tpu_sc as plsc

N = 1000
NBP = 1024
TBL = N * NBP
NUM_SAMPLES = 2_000_000
NC, NS, L = 2, 16, 16
NW = NC * NS
NB = NUM_SAMPLES // 128
BPW = NB // NW
REM = NB - BPW * NW
CHB = 32
N_FULL = BPW // CHB
TAILB = BPW - N_FULL * CHB + REM
CH = CHB * 128
CBL = TAILB * 128
ZB = 8000
Z_SPAN = TBL // NS
DUMMY = (7 << 10) | 127


def _hist_body(inp, ones_c, zeros_c, out, table, inbuf0, inbuf1, cbuf0,
               cbuf1, ones1d, cbuf_l, ones_l, sem_in0, sem_in1, sem_s0,
               sem_s1, sem_l, sem_z):
    cid = lax.axis_index("c")
    sid = lax.axis_index("s")
    wid = cid * NS + sid
    base = wid * BPW
    iota = lax.iota(jnp.int32, L)
    dummy16 = jnp.full((L,), DUMMY, jnp.int32)
    inbufs = [inbuf0, inbuf1]
    cbufs = [cbuf0, cbuf1]
    sem_in = [sem_in0, sem_in1]
    sem_s = [sem_s0, sem_s1]

    fills = [
        pltpu.async_copy(ones_c, ones1d, sem_z),
        pltpu.async_copy(ones_c.at[pl.ds(0, CBL)], ones_l, sem_z),
        pltpu.async_copy(zeros_c, table.at[pl.ds(sid * Z_SPAN, Z_SPAN)], sem_z),
    ]
    pending_in = pltpu.async_copy(inp.at[pl.ds(base, CHB)], inbufs[0], sem_in[0])
    for c in fills:
        c.wait()
    plsc.subcore_barrier()

    def tiled_bin(va, vb):
        return (
            ((va >> 3) << 13)
            | ((vb >> 7) << 10)
            | ((va & 7) << 7)
            | (vb & 127)
        )

    def compute_blocks(src, cdst):
        @plsc.parallel_loop(0, CHB, unroll=2)
        def blk_body(blk):
            for gg in range(8):
                va = src[blk, 0, pl.ds(gg * L, L)]
                vb = src[blk, 1, pl.ds(gg * L, L)]
                cdst[pl.ds(blk * 128 + gg * L, L)] = tiled_bin(va, vb)

    nreal = jnp.where(wid == NW - 1, TAILB, TAILB - REM)

    def compute_tail(src):
        @plsc.parallel_loop(0, TAILB)
        def blk_body_l(blk):
            for gg in range(8):
                va = src[blk, 0, pl.ds(gg * L, L)]
                vb = src[blk, 1, pl.ds(gg * L, L)]
                c = jnp.where(blk < nreal, tiled_bin(va, vb), dummy16)
                cbuf_l[pl.ds(blk * 128 + gg * L, L)] = c

    prev_scat = None
    for ch in range(N_FULL + 1):
        cur = ch & 1
        nxt = 1 - cur
        next_in = None
        if ch + 1 < N_FULL:
            next_in = pltpu.async_copy(
                inp.at[pl.ds(base + (ch + 1) * CHB, CHB)], inbufs[nxt], sem_in[nxt]
            )
        elif ch + 1 == N_FULL:
            next_in = pltpu.async_copy(
                inp.at[pl.ds(base + N_FULL * CHB, TAILB)],
                inbufs[nxt].at[pl.ds(0, TAILB)],
                sem_in[nxt],
            )
        pending_in.wait()
        if ch < N_FULL:
            compute_blocks(inbufs[cur], cbufs[cur])
            if prev_scat is not None:
                prev_scat.wait()
            prev_scat = pltpu.async_copy(
                ones1d, table.at[cbufs[cur]], sem_s[cur], add=True
            )
        else:
            compute_tail(inbufs[cur])
            if prev_scat is not None:
                prev_scat.wait()
            prev_scat = pltpu.async_copy(ones_l, table.at[cbuf_l], sem_l, add=True)
        pending_in = next_in

    prev_scat.wait()

    plsc.subcore_barrier()
    pltpu.sync_copy(
        table.at[pl.ds(sid * Z_SPAN, Z_SPAN)],
        out.at[pl.ds(cid * TBL + sid * Z_SPAN, Z_SPAN)],
    )


_sc_hist = functools.partial(
    pl.kernel,
    out_type=jax.ShapeDtypeStruct((NC * TBL,), jnp.float32),
    mesh=plsc.VectorSubcoreMesh(core_axis_name="c", subcore_axis_name="s"),
    compiler_params=pltpu.CompilerParams(needs_layout_passes=False),
    scratch_types=[
        pltpu.VMEM_SHARED((TBL,), jnp.float32),
        pltpu.VMEM((CHB, 2, 128), jnp.int32),
        pltpu.VMEM((CHB, 2, 128), jnp.int32),
        pltpu.VMEM((CH,), jnp.int32),
        pltpu.VMEM((CH,), jnp.int32),
        pltpu.VMEM((CH,), jnp.float32),
        pltpu.VMEM((CBL,), jnp.int32),
        pltpu.VMEM((CBL,), jnp.float32),
        pltpu.SemaphoreType.DMA,
        pltpu.SemaphoreType.DMA,
        pltpu.SemaphoreType.DMA,
        pltpu.SemaphoreType.DMA,
        pltpu.SemaphoreType.DMA,
        pltpu.SemaphoreType.DMA,
    ],
)(_hist_body)


def _finalize_body(p_ref, o_ref, a_ref):
    s = p_ref[0] + p_ref[1]
    v = s[:, :N]
    rs = jnp.sum(v, axis=1, keepdims=True)
    a_ref[...] = rs * (1.0 / NUM_SAMPLES)
    o_ref[...] = v / jnp.maximum(rs, 1.0)


_FROWS = 200


def _finalize(p3):
    return pl.pallas_call(
        _finalize_body,
        grid=(N // _FROWS,),
        in_specs=[pl.BlockSpec((NC, _FROWS, NBP), lambda i: (0, i, 0))],
        out_specs=[
            pl.BlockSpec((_FROWS, N), lambda i: (i, 0)),
            pl.BlockSpec((_FROWS, 1), lambda i: (i, 0)),
        ],
        out_shape=[
            jax.ShapeDtypeStruct((N, N), jnp.float32),
            jax.ShapeDtypeStruct((N, 1), jnp.float32),
        ],
    )(p3)


def kernel(inputs, vals):
    del vals
    view = inputs.reshape(NB, 128, 2).transpose(0, 2, 1)
    part = _sc_hist(
        view,
        jnp.ones((CH,), jnp.float32),
        jnp.zeros((Z_SPAN,), jnp.float32),
    )
    p3 = (
        part.reshape(NC, N // 8, 8, NBP // 128, 128)
        .transpose(0, 1, 3, 2, 4)
        .reshape(NC, N, NBP)
    )
    pi_b_a, pi_a = _finalize(p3)
    return pi_a.reshape(N), pi_b_a


# speedup vs baseline: 1.0067x; 1.0014x over previous
import functools

import jax
import jax.numpy as jnp
from jax import lax
from jax.experimental import pallas as pl
from jax.experimental.pallas import tpu as pltpu
from jax.experimental.pallas import tpu_sc as plsc

N = 1000
NBP = 1024
TBL = N * NBP
NUM_SAMPLES = 2_000_000
NC, NS, L = 2, 16, 16
NW = NC * NS
NB = NUM_SAMPLES // 128
BPW = NB // NW
REM = NB - BPW * NW
CHB = 32
N_FULL = BPW // CHB
TAILB = BPW - N_FULL * CHB + REM
CH = CHB * 128
CBL = TAILB * 128
Z_SPAN = TBL // NS
DUMMY = (7 << 10) | 127


def _hist_body(inp, ones_c, zeros_c, out, table, inbuf0, inbuf1, cbuf0,
               cbuf1, ones1d, cbuf_l, ones_l, sem_in0, sem_in1, sem_s0,
               sem_s1, sem_l, sem_z):
    cid = lax.axis_index("c")
    sid = lax.axis_index("s")
    wid = cid * NS + sid
    base = wid * BPW
    dummy16 = jnp.full((L,), DUMMY, jnp.int32)
    inbufs = [inbuf0, inbuf1]
    cbufs = [cbuf0, cbuf1]
    sem_in = [sem_in0, sem_in1]
    sem_s = [sem_s0, sem_s1]

    fills = [
        pltpu.async_copy(ones_c, ones1d, sem_z),
        pltpu.async_copy(ones_c.at[pl.ds(0, CBL)], ones_l, sem_z),
        pltpu.async_copy(zeros_c, table.at[pl.ds(sid * Z_SPAN, Z_SPAN)], sem_z),
    ]
    pending_in = pltpu.async_copy(inp.at[pl.ds(base, CHB)], inbufs[0], sem_in[0])
    for c in fills:
        c.wait()
    plsc.subcore_barrier()

    def tiled_bin(va, vb):
        return (
            ((va >> 3) << 13)
            | ((vb >> 7) << 10)
            | ((va & 7) << 7)
            | (vb & 127)
        )

    def compute_blocks(src, cdst):
        @plsc.parallel_loop(0, CHB, unroll=2)
        def blk_body(blk):
            for gg in range(8):
                va = src[blk, 0, pl.ds(gg * L, L)]
                vb = src[blk, 1, pl.ds(gg * L, L)]
                cdst[pl.ds(blk * 128 + gg * L, L)] = tiled_bin(va, vb)

    nreal = jnp.where(wid == NW - 1, TAILB, TAILB - REM)

    def compute_tail(src):
        @plsc.parallel_loop(0, TAILB)
        def blk_body_l(blk):
            for gg in range(8):
                va = src[blk, 0, pl.ds(gg * L, L)]
                vb = src[blk, 1, pl.ds(gg * L, L)]
                c = jnp.where(blk < nreal, tiled_bin(va, vb), dummy16)
                cbuf_l[pl.ds(blk * 128 + gg * L, L)] = c

    prev_scat = None
    for ch in range(N_FULL + 1):
        cur = ch & 1
        nxt = 1 - cur
        next_in = None
        if ch + 1 < N_FULL:
            next_in = pltpu.async_copy(
                inp.at[pl.ds(base + (ch + 1) * CHB, CHB)], inbufs[nxt], sem_in[nxt]
            )
        elif ch + 1 == N_FULL:
            next_in = pltpu.async_copy(
                inp.at[pl.ds(base + N_FULL * CHB, TAILB)],
                inbufs[nxt].at[pl.ds(0, TAILB)],
                sem_in[nxt],
            )
        pending_in.wait()
        if ch < N_FULL:
            compute_blocks(inbufs[cur], cbufs[cur])
            if prev_scat is not None:
                prev_scat.wait()
            prev_scat = pltpu.async_copy(
                ones1d, table.at[cbufs[cur]], sem_s[cur], add=True
            )
        else:
            compute_tail(inbufs[cur])
            if prev_scat is not None:
                prev_scat.wait()
            prev_scat = pltpu.async_copy(ones_l, table.at[cbuf_l], sem_l, add=True)
        pending_in = next_in

    prev_scat.wait()

    plsc.subcore_barrier()
    pltpu.sync_copy(
        table.at[pl.ds(sid * Z_SPAN, Z_SPAN)],
        out.at[pl.ds(cid * TBL + sid * Z_SPAN, Z_SPAN)],
    )


_sc_hist = functools.partial(
    pl.kernel,
    out_type=jax.ShapeDtypeStruct((NC * TBL,), jnp.float32),
    mesh=plsc.VectorSubcoreMesh(core_axis_name="c", subcore_axis_name="s"),
    compiler_params=pltpu.CompilerParams(needs_layout_passes=False),
    scratch_types=[
        pltpu.VMEM_SHARED((TBL,), jnp.float32),
        pltpu.VMEM((CHB, 2, 128), jnp.int32),
        pltpu.VMEM((CHB, 2, 128), jnp.int32),
        pltpu.VMEM((CH,), jnp.int32),
        pltpu.VMEM((CH,), jnp.int32),
        pltpu.VMEM((CH,), jnp.float32),
        pltpu.VMEM((CBL,), jnp.int32),
        pltpu.VMEM((CBL,), jnp.float32),
        pltpu.SemaphoreType.DMA,
        pltpu.SemaphoreType.DMA,
        pltpu.SemaphoreType.DMA,
        pltpu.SemaphoreType.DMA,
        pltpu.SemaphoreType.DMA,
        pltpu.SemaphoreType.DMA,
    ],
)(_hist_body)


def _finalize_body(p_ref, o_ref, a_ref):
    s = p_ref[0] + p_ref[1]
    v = s[:, :N]
    rs = jnp.sum(v, axis=1, keepdims=True)
    a_ref[...] = rs * (1.0 / NUM_SAMPLES)
    o_ref[...] = v / jnp.maximum(rs, 1.0)


_FROWS = 200


def _finalize(p3):
    return pl.pallas_call(
        _finalize_body,
        grid=(N // _FROWS,),
        in_specs=[pl.BlockSpec((NC, _FROWS, NBP), lambda i: (0, i, 0))],
        out_specs=[
            pl.BlockSpec((_FROWS, N), lambda i: (i, 0)),
            pl.BlockSpec((_FROWS, 1), lambda i: (i, 0)),
        ],
        out_shape=[
            jax.ShapeDtypeStruct((N, N), jnp.float32),
            jax.ShapeDtypeStruct((N, 1), jnp.float32),
        ],
    )(p3)


def kernel(inputs, vals):
    del vals
    view = inputs.reshape(NB, 128, 2).transpose(0, 2, 1)
    part = _sc_hist(
        view,
        jnp.ones((CH,), jnp.float32),
        jnp.zeros((Z_SPAN,), jnp.float32),
    )
    p3 = (
        part.reshape(NC, N // 8, 8, NBP // 128, 128)
        .transpose(0, 1, 3, 2, 4)
        .reshape(NC, N, NBP)
    )
    pi_b_a, pi_a = _finalize(p3)
    return pi_a.reshape(N), pi_b_a
